# baseline (speedup 1.0000x reference)
import functools

import jax
import jax.numpy as jnp
import numpy as np
from jax import lax
from jax.experimental import pallas as pl
from jax.experimental.pallas import tpu as pltpu
from jax.experimental.pallas import tpu_sc as plsc

A_MAX = 7.25
S_MAX = 9.25

B = 128
J = 22
F = 6600
FLAT = F * J
NW = 32
BPW = B // NW
NCH = 5
CHUNK = FLAT // NCH
APC = CHUNK // 176
FP = 6608

def _field_xy():
    x = np.linspace(0.5, 119.5, 120, dtype=np.float32)
    y = np.linspace(-0.5, 53.5, 55, dtype=np.float32)
    y[0] = -0.2
    yy, xx = np.meshgrid(y, x, indexing="ij")
    fx = np.zeros(FP, np.float32)
    fy = np.zeros(FP, np.float32)
    fx[:F] = xx.reshape(-1)
    fy[:F] = yy.reshape(-1)
    return fx, fy


def _rsqrt(x):
    i = plsc.bitcast(x, jnp.int32)
    i = jnp.int32(0x5F3759DF) - (i >> 1)
    y = plsc.bitcast(i, jnp.float32)
    xh = x * 0.5
    y = y * (1.5 - xh * y * y)
    y = y * (1.5 - xh * y * y)
    return y


def _sc_body(params_hbm, fx_hbm, fy_hbm, out_hbm,
             pv, fxv, fyv, buf0, buf1, sem0, sem1):
    wid = lax.axis_index("s") * 2 + lax.axis_index("c")
    pltpu.sync_copy(fx_hbm, fxv)
    pltpu.sync_copy(fy_hbm, fyv)
    pltpu.sync_copy(
        params_hbm.at[pl.ds(pl.multiple_of(wid * (BPW * 128), BPW * 128),
                            BPW * 128)], pv)
    bufs = (buf0, buf1)
    sems = (sem0, sem1)
    lane = lax.iota(jnp.int32, 16)
    jk, dk = [], []
    for k in range(11):
        flat = lane + (16 * k)
        jk.append(flat % 22)
        dk.append(flat // 22)

    def outer(g, carry):
        for slot in range(2):
            m = g * 2 + slot
            bi = m // NCH
            c = m % NCH
            b = wid * BPW + bi

            @pl.when(g > 0)
            def _wait():
                pltpu.make_async_copy(
                    bufs[slot], out_hbm.at[pl.ds(0, CHUNK)], sems[slot]).wait()

            boff = bi * 128
            jkb = [v + boff for v in jk]
            jkyb = [v + 32 for v in jkb]
            jkvxb = [v + 64 for v in jkb]
            jkvyb = [v + 96 for v in jkb]
            buf = bufs[slot]
            f0c = c * 1320

            def body_a(a, cc, jkb=jkb, jkyb=jkyb, jkvxb=jkvxb, jkvyb=jkvyb,
                       f0c=f0c, buf=buf):
                base = a * 176
                f0 = f0c + a * 8
                for k in range(11):
                    fidx = dk[k] + f0
                    fxg = plsc.load_gather(fxv, [fidx])
                    fyg = plsc.load_gather(fyv, [fidx])
                    x = plsc.load_gather(pv, [jkb[k]])
                    y = plsc.load_gather(pv, [jkyb[k]])
                    vx = plsc.load_gather(pv, [jkvxb[k]])
                    vy = plsc.load_gather(pv, [jkvyb[k]])
                    dx = fxg - x
                    dy = fyg - y
                    d2 = dx * dx + dy * dy
                    r = _rsqrt(d2)
                    d = d2 * r
                    s0 = jnp.clip((dx * vx + dy * vy) * r, -S_MAX, S_MAX)
                    u = s0 * (1.0 / A_MAX)
                    t1 = (S_MAX / A_MAX) - u
                    dlt = t1 * (0.5 * s0 + 0.5 * S_MAX)
                    q = u * u + (2.0 / A_MAX) * d
                    t2 = q * _rsqrt(q) - u
                    tl = jnp.where(dlt > d, t2, t1)
                    dl = jnp.maximum(jnp.minimum(dlt, d), 0.0)
                    t = tl + (d - dl) * (1.0 / S_MAX)
                    buf[pl.ds(base + k * 16, 16)] = t
                return cc

            lax.fori_loop(0, APC, body_a, 0)
            off = pl.multiple_of(b * FLAT + c * CHUNK, 16)
            pltpu.async_copy(buf, out_hbm.at[pl.ds(off, CHUNK)], sems[slot])
        return carry

    lax.fori_loop(0, (BPW * NCH) // 2, outer, 0)
    for slot in range(2):
        pltpu.make_async_copy(
            bufs[slot], out_hbm.at[pl.ds(0, CHUNK)], sems[slot]).wait()


@jax.jit
def _run_sc(params, fx, fy):
    mesh = plsc.VectorSubcoreMesh(core_axis_name="c", subcore_axis_name="s")
    fn = pl.kernel(
        _sc_body,
        out_type=jax.ShapeDtypeStruct((B * FLAT,), jnp.float32),
        mesh=mesh,
        scratch_types=[
            pltpu.VMEM((BPW * 128,), jnp.float32),
            pltpu.VMEM((FP,), jnp.float32),
            pltpu.VMEM((FP,), jnp.float32),
            pltpu.VMEM((CHUNK,), jnp.float32),
            pltpu.VMEM((CHUNK,), jnp.float32),
            pltpu.SemaphoreType.DMA,
            pltpu.SemaphoreType.DMA,
        ],
        compiler_params=pltpu.CompilerParams(needs_layout_passes=False),
    )
    return fn(params, fx, fy)


def kernel(frame):
    p = jnp.zeros((B, 4, 32), jnp.float32)
    p = p.at[:, 0, :J].set(frame[:, :, 1])
    p = p.at[:, 1, :J].set(frame[:, :, 2])
    p = p.at[:, 2, :J].set(frame[:, :, 3])
    p = p.at[:, 3, :J].set(frame[:, :, 4])
    fx_np, fy_np = _field_xy()
    out = _run_sc(p.reshape(B * 128), jnp.asarray(fx_np), jnp.asarray(fy_np))
    return out.reshape(B, F, J)


# speedup vs baseline: 1.4799x; 1.4799x over previous
import functools

import jax
import jax.numpy as jnp
import numpy as np
from jax import lax
from jax.experimental import pallas as pl
from jax.experimental.pallas import tpu as pltpu
from jax.experimental.pallas import tpu_sc as plsc

A_MAX = 7.25
S_MAX = 9.25

B = 128
J = 22
F = 6600
FLAT = F * J
NW = 32
BPW = B // NW
NCH = 5
CHUNK = FLAT // NCH
APC = CHUNK // 176
FP = 6608

def _field_xy():
    x = np.linspace(0.5, 119.5, 120, dtype=np.float32)
    y = np.linspace(-0.5, 53.5, 55, dtype=np.float32)
    y[0] = -0.2
    yy, xx = np.meshgrid(y, x, indexing="ij")
    fx = np.zeros(FP, np.float32)
    fy = np.zeros(FP, np.float32)
    fx[:F] = xx.reshape(-1)
    fy[:F] = yy.reshape(-1)
    return fx, fy


def _rsqrt(x):
    i = plsc.bitcast(x, jnp.int32)
    i = jnp.int32(0x5F3759DF) - (i >> 1)
    y = plsc.bitcast(i, jnp.float32)
    xh = x * 0.5
    y = y * (1.5 - xh * y * y)
    y = y * (1.5 - xh * y * y)
    return y


def _sc_body(params_hbm, fx_hbm, fy_hbm, out_hbm,
             pv, fxv, fyv, buf0, buf1, sem0, sem1):
    wid = lax.axis_index("s") * 2 + lax.axis_index("c")
    pltpu.sync_copy(fx_hbm, fxv)
    pltpu.sync_copy(fy_hbm, fyv)
    pltpu.sync_copy(
        params_hbm.at[pl.ds(pl.multiple_of(wid * (BPW * 128), BPW * 128),
                            BPW * 128)], pv)
    bufs = (buf0, buf1)
    sems = (sem0, sem1)
    lane = lax.iota(jnp.int32, 16)
    jk, dk = [], []
    for k in range(11):
        flat = lane + (16 * k)
        jk.append(flat % 22)
        dk.append(flat // 22)

    def outer(g, carry):
        for slot in range(2):
            m = g * 2 + slot
            bi = m // NCH
            c = m % NCH
            b = wid * BPW + bi

            @pl.when(g > 0)
            def _wait():
                pltpu.make_async_copy(
                    bufs[slot], out_hbm.at[pl.ds(0, CHUNK)], sems[slot]).wait()

            boff = bi * 128
            jkb = [v + boff for v in jk]
            jkyb = [v + 32 for v in jkb]
            jkvxb = [v + 64 for v in jkb]
            jkvyb = [v + 96 for v in jkb]
            buf = bufs[slot]
            f0c = c * 1320

            def body_a(a, jkb=jkb, jkyb=jkyb, jkvxb=jkvxb, jkvyb=jkvyb,
                       f0c=f0c, buf=buf):
                base = a * 176
                f0 = f0c + a * 8
                for k in range(11):
                    fidx = dk[k] + f0
                    fxg = plsc.load_gather(fxv, [fidx])
                    fyg = plsc.load_gather(fyv, [fidx])
                    x = plsc.load_gather(pv, [jkb[k]])
                    y = plsc.load_gather(pv, [jkyb[k]])
                    vx = plsc.load_gather(pv, [jkvxb[k]])
                    vy = plsc.load_gather(pv, [jkvyb[k]])
                    dx = fxg - x
                    dy = fyg - y
                    d2 = dx * dx + dy * dy
                    r = _rsqrt(d2)
                    d = d2 * r
                    s0 = jnp.clip((dx * vx + dy * vy) * r, -S_MAX, S_MAX)
                    u = s0 * (1.0 / A_MAX)
                    t1 = (S_MAX / A_MAX) - u
                    dlt = t1 * (0.5 * s0 + 0.5 * S_MAX)
                    q = u * u + (2.0 / A_MAX) * d
                    t2 = q * _rsqrt(q) - u
                    tl = jnp.where(dlt > d, t2, t1)
                    dl = jnp.maximum(jnp.minimum(dlt, d), 0.0)
                    t = tl + (d - dl) * (1.0 / S_MAX)
                    buf[pl.ds(base + k * 16, 16)] = t

            plsc.parallel_loop(0, APC, 1, unroll=2)(body_a)
            off = pl.multiple_of(b * FLAT + c * CHUNK, 16)
            pltpu.async_copy(buf, out_hbm.at[pl.ds(off, CHUNK)], sems[slot])
        return carry

    lax.fori_loop(0, (BPW * NCH) // 2, outer, 0)
    for slot in range(2):
        pltpu.make_async_copy(
            bufs[slot], out_hbm.at[pl.ds(0, CHUNK)], sems[slot]).wait()


@jax.jit
def _run_sc(params, fx, fy):
    mesh = plsc.VectorSubcoreMesh(core_axis_name="c", subcore_axis_name="s")
    fn = pl.kernel(
        _sc_body,
        out_type=jax.ShapeDtypeStruct((B * FLAT,), jnp.float32),
        mesh=mesh,
        scratch_types=[
            pltpu.VMEM((BPW * 128,), jnp.float32),
            pltpu.VMEM((FP,), jnp.float32),
            pltpu.VMEM((FP,), jnp.float32),
            pltpu.VMEM((CHUNK,), jnp.float32),
            pltpu.VMEM((CHUNK,), jnp.float32),
            pltpu.SemaphoreType.DMA,
            pltpu.SemaphoreType.DMA,
        ],
        compiler_params=pltpu.CompilerParams(needs_layout_passes=False),
    )
    return fn(params, fx, fy)


def kernel(frame):
    p = jnp.zeros((B, 4, 32), jnp.float32)
    p = p.at[:, 0, :J].set(frame[:, :, 1])
    p = p.at[:, 1, :J].set(frame[:, :, 2])
    p = p.at[:, 2, :J].set(frame[:, :, 3])
    p = p.at[:, 3, :J].set(frame[:, :, 4])
    fx_np, fy_np = _field_xy()
    out = _run_sc(p.reshape(B * 128), jnp.asarray(fx_np), jnp.asarray(fy_np))
    return out.reshape(B, F, J)


# speedup vs baseline: 4.9142x; 3.3206x over previous
import jax
import jax.numpy as jnp
import numpy as np
from jax.experimental import pallas as pl

A_MAX = 7.25
S_MAX = 9.25
B = 128
J = 22
F = 6600


def _field_xy():
    x = np.linspace(0.5, 119.5, 120, dtype=np.float32)
    y = np.linspace(-0.5, 53.5, 55, dtype=np.float32)
    y[0] = -0.2
    yy, xx = np.meshgrid(y, x, indexing="ij")
    return xx.reshape(1, F), yy.reshape(1, F)


def _body(x_ref, y_ref, vx_ref, vy_ref, fx_ref, fy_ref, out_ref):
    x = x_ref[0]
    y = y_ref[0]
    vx = vx_ref[0]
    vy = vy_ref[0]
    fx = fx_ref[...]
    fy = fy_ref[...]
    dx = fx - x
    dy = fy - y
    d2 = dx * dx + dy * dy
    r = jax.lax.rsqrt(d2)
    d = d2 * r
    s0 = jnp.clip((dx * vx + dy * vy) * r, -S_MAX, S_MAX)
    u = s0 * (1.0 / A_MAX)
    t1 = (S_MAX / A_MAX) - u
    dlt = t1 * (0.5 * s0 + 0.5 * S_MAX)
    q = u * u + (2.0 / A_MAX) * d
    t2 = q * jax.lax.rsqrt(q) - u
    tl = jnp.where(dlt > d, t2, t1)
    dl = jnp.maximum(jnp.minimum(dlt, d), 0.0)
    t = tl + (d - dl) * (1.0 / S_MAX)
    out_ref[0] = t.T


@jax.jit
def _run(xp, yp, vxp, vyp, fx, fy):
    return pl.pallas_call(
        _body,
        grid=(B,),
        in_specs=[
            pl.BlockSpec((1, J, 1), lambda b: (b, 0, 0)),
            pl.BlockSpec((1, J, 1), lambda b: (b, 0, 0)),
            pl.BlockSpec((1, J, 1), lambda b: (b, 0, 0)),
            pl.BlockSpec((1, J, 1), lambda b: (b, 0, 0)),
            pl.BlockSpec((1, F), lambda b: (0, 0)),
            pl.BlockSpec((1, F), lambda b: (0, 0)),
        ],
        out_specs=pl.BlockSpec((1, F, J), lambda b: (b, 0, 0)),
        out_shape=jax.ShapeDtypeStruct((B, F, J), jnp.float32),
    )(xp, yp, vxp, vyp, fx, fy)


def kernel(frame):
    xp = frame[:, :, 1:2]
    yp = frame[:, :, 2:3]
    vxp = frame[:, :, 3:4]
    vyp = frame[:, :, 4:5]
    fx_np, fy_np = _field_xy()
    return _run(xp, yp, vxp, vyp, jnp.asarray(fx_np), jnp.asarray(fy_np))


# speedup vs baseline: 4.9502x; 1.0073x over previous
import jax
import jax.numpy as jnp
import numpy as np
from jax.experimental import pallas as pl

A_MAX = 7.25
S_MAX = 9.25
B = 128
J = 22
F = 6600
NY = 55
NX = 120
LX = NX * J


def _consts():
    l = np.arange(LX)
    fx = (0.5 + (l // J)).astype(np.float32).reshape(1, LX)
    y = np.linspace(-0.5, 53.5, NY, dtype=np.float32)
    y[0] = -0.2
    fy = y.reshape(1, NY, 1)
    return fx, fy


def _body(x_ref, y_ref, vx_ref, vy_ref, fx_ref, fy_ref, out_ref):
    x = x_ref[0]
    y = y_ref[0]
    vx = vx_ref[0]
    vy = vy_ref[0]
    fx = fx_ref[...]
    fy = fy_ref[0]
    dx = fx - x
    dy = fy - y
    dx2 = dx * dx
    d2 = dx2 + dy * dy
    r = jax.lax.rsqrt(d2)
    d = d2 * r
    s0 = jnp.clip((dx * vx + dy * vy) * r, -S_MAX, S_MAX)
    u = s0 * (1.0 / A_MAX)
    t1 = (S_MAX / A_MAX) - u
    dlt = t1 * (0.5 * s0 + 0.5 * S_MAX)
    q = u * u + (2.0 / A_MAX) * d
    t2 = q * jax.lax.rsqrt(q) - u
    tl = jnp.where(dlt > d, t2, t1)
    dl = jnp.maximum(jnp.minimum(dlt, d), 0.0)
    out_ref[0] = tl + (d - dl) * (1.0 / S_MAX)


@jax.jit
def _run(xe, ye, vxe, vye, fx, fy):
    return pl.pallas_call(
        _body,
        grid=(B,),
        in_specs=[
            pl.BlockSpec((1, 1, LX), lambda b: (b, 0, 0)),
            pl.BlockSpec((1, 1, LX), lambda b: (b, 0, 0)),
            pl.BlockSpec((1, 1, LX), lambda b: (b, 0, 0)),
            pl.BlockSpec((1, 1, LX), lambda b: (b, 0, 0)),
            pl.BlockSpec((1, LX), lambda b: (0, 0)),
            pl.BlockSpec((1, NY, 1), lambda b: (0, 0, 0)),
        ],
        out_specs=pl.BlockSpec((1, NY, LX), lambda b: (b, 0, 0)),
        out_shape=jax.ShapeDtypeStruct((B, NY, LX), jnp.float32),
    )(xe, ye, vxe, vye, fx, fy)


def kernel(frame):
    xe = jnp.tile(frame[:, :, 1], (1, NX))[:, None, :]
    ye = jnp.tile(frame[:, :, 2], (1, NX))[:, None, :]
    vxe = jnp.tile(frame[:, :, 3], (1, NX))[:, None, :]
    vye = jnp.tile(frame[:, :, 4], (1, NX))[:, None, :]
    fx_np, fy_np = _consts()
    out = _run(xe, ye, vxe, vye, jnp.asarray(fx_np), jnp.asarray(fy_np))
    return out.reshape(B, F, J)


# speedup vs baseline: 5.4812x; 1.1073x over previous
import jax
import jax.numpy as jnp
import numpy as np
from jax.experimental import pallas as pl
from jax.experimental.pallas import tpu as pltpu

A_MAX = 7.25
S_MAX = 9.25
B = 128
J = 22
F = 6600
NBUF = 4
NG = B // NBUF


def _field_xy():
    x = np.linspace(0.5, 119.5, 120, dtype=np.float32)
    y = np.linspace(-0.5, 53.5, 55, dtype=np.float32)
    y[0] = -0.2
    yy, xx = np.meshgrid(y, x, indexing="ij")
    return xx.reshape(1, F), yy.reshape(1, F)


def _compute(x, y, vx, vy, fx, fy):
    dx = fx - x
    dy = fy - y
    d2 = dx * dx + dy * dy
    r = jax.lax.rsqrt(d2)
    d = d2 * r
    s0 = jnp.clip((dx * vx + dy * vy) * r, -S_MAX, S_MAX)
    s02 = s0 * s0
    dlt = (S_MAX * S_MAX / (2.0 * A_MAX)) - s02 * (0.5 / A_MAX)
    qq = s02 + (2.0 * A_MAX) * d
    sq = qq * jax.lax.rsqrt(qq)
    us = s0 * (1.0 / A_MAX)
    t2 = sq * (1.0 / A_MAX) - us
    t1 = (S_MAX / A_MAX) - us
    tl = jnp.where(dlt > d, t2, t1)
    dd = jnp.maximum(d - jnp.maximum(dlt, 0.0), 0.0)
    return tl + dd * (1.0 / S_MAX)


def _body(x_ref, y_ref, vx_ref, vy_ref, fx_ref, fy_ref, out_hbm,
          buf, sem0, sem1, sem2, sem3):
    g = pl.program_id(0)
    sems = (sem0, sem1, sem2, sem3)
    fx = fx_ref[...]
    fy = fy_ref[...]
    for s in range(NBUF):
        b = g * NBUF + s

        @pl.when(g > 0)
        def _wait(s=s):
            pltpu.make_async_copy(
                buf.at[s], out_hbm.at[pl.ds(0, 1)], sems[s]).wait()

        t = _compute(x_ref[s], y_ref[s], vx_ref[s], vy_ref[s], fx, fy)
        buf[s] = t.T[None]
        pltpu.async_copy(buf.at[s], out_hbm.at[pl.ds(b, 1)], sems[s])

    @pl.when(g == NG - 1)
    def _drain():
        for s in range(NBUF):
            pltpu.make_async_copy(
                buf.at[s], out_hbm.at[pl.ds(0, 1)], sems[s]).wait()


@jax.jit
def _run(xp, yp, vxp, vyp, fx, fy):
    return pl.pallas_call(
        _body,
        grid=(NG,),
        in_specs=[
            pl.BlockSpec((NBUF, J, 1), lambda g: (g, 0, 0)),
            pl.BlockSpec((NBUF, J, 1), lambda g: (g, 0, 0)),
            pl.BlockSpec((NBUF, J, 1), lambda g: (g, 0, 0)),
            pl.BlockSpec((NBUF, J, 1), lambda g: (g, 0, 0)),
            pl.BlockSpec((1, F), lambda g: (0, 0)),
            pl.BlockSpec((1, F), lambda g: (0, 0)),
        ],
        out_specs=pl.BlockSpec(memory_space=pltpu.MemorySpace.HBM),
        out_shape=jax.ShapeDtypeStruct((B, F, J), jnp.float32),
        scratch_shapes=[
            pltpu.VMEM((NBUF, 1, F, J), jnp.float32),
            pltpu.SemaphoreType.DMA,
            pltpu.SemaphoreType.DMA,
            pltpu.SemaphoreType.DMA,
            pltpu.SemaphoreType.DMA,
        ],
    )(xp, yp, vxp, vyp, fx, fy)


def kernel(frame):
    xp = frame[:, :, 1:2]
    yp = frame[:, :, 2:3]
    vxp = frame[:, :, 3:4]
    vyp = frame[:, :, 4:5]
    fx_np, fy_np = _field_xy()
    return _run(xp, yp, vxp, vyp, jnp.asarray(fx_np), jnp.asarray(fy_np))


# speedup vs baseline: 5.4824x; 1.0002x over previous
import jax
import jax.numpy as jnp
import numpy as np
from jax.experimental import pallas as pl
from jax.experimental.pallas import tpu as pltpu

A_MAX = 7.25
S_MAX = 9.25
B = 128
J = 22
F = 6600
NBUF = 4
NG = B // NBUF
NSPL = 4
FS = F // NSPL


def _field_xy():
    x = np.linspace(0.5, 119.5, 120, dtype=np.float32)
    y = np.linspace(-0.5, 53.5, 55, dtype=np.float32)
    y[0] = -0.2
    yy, xx = np.meshgrid(y, x, indexing="ij")
    return xx.reshape(1, F), yy.reshape(1, F)


def _compute(x, y, vx, vy, fx, fy):
    dx = fx - x
    dy = fy - y
    d2 = dx * dx + dy * dy
    r = jax.lax.rsqrt(d2)
    d = d2 * r
    s0 = jnp.clip((dx * vx + dy * vy) * r, -S_MAX, S_MAX)
    s02 = s0 * s0
    dlt = (S_MAX * S_MAX / (2.0 * A_MAX)) - s02 * (0.5 / A_MAX)
    qq = s02 + (2.0 * A_MAX) * d
    sq = qq * jax.lax.rsqrt(qq)
    us = s0 * (1.0 / A_MAX)
    t2 = sq * (1.0 / A_MAX) - us
    t1 = (S_MAX / A_MAX) - us
    tl = jnp.where(dlt > d, t2, t1)
    dd = jnp.maximum(d - jnp.maximum(dlt, 0.0), 0.0)
    return tl + dd * (1.0 / S_MAX)


def _body(x_ref, y_ref, vx_ref, vy_ref, fx_ref, fy_ref, out_hbm, buf, *sems):
    g = pl.program_id(0)
    fx = fx_ref[...]
    fy = fy_ref[...]
    for s in range(NBUF):
        b = g * NBUF + s

        @pl.when(g > 0)
        def _wait(s=s):
            for h in range(NSPL):
                pltpu.make_async_copy(
                    buf.at[s, pl.ds(h * FS, FS)],
                    out_hbm.at[0, pl.ds(h * FS, FS)],
                    sems[s * NSPL + h]).wait()

        t = _compute(x_ref[s], y_ref[s], vx_ref[s], vy_ref[s], fx, fy)
        buf[s] = t.T
        for h in range(NSPL):
            pltpu.async_copy(
                buf.at[s, pl.ds(h * FS, FS)],
                out_hbm.at[b, pl.ds(h * FS, FS)],
                sems[s * NSPL + h])

    @pl.when(g == NG - 1)
    def _drain():
        for s in range(NBUF):
            for h in range(NSPL):
                pltpu.make_async_copy(
                    buf.at[s, pl.ds(h * FS, FS)],
                    out_hbm.at[0, pl.ds(h * FS, FS)],
                    sems[s * NSPL + h]).wait()


@jax.jit
def _run(xp, yp, vxp, vyp, fx, fy):
    return pl.pallas_call(
        _body,
        grid=(NG,),
        in_specs=[
            pl.BlockSpec((NBUF, J, 1), lambda g: (g, 0, 0)),
            pl.BlockSpec((NBUF, J, 1), lambda g: (g, 0, 0)),
            pl.BlockSpec((NBUF, J, 1), lambda g: (g, 0, 0)),
            pl.BlockSpec((NBUF, J, 1), lambda g: (g, 0, 0)),
            pl.BlockSpec((1, F), lambda g: (0, 0)),
            pl.BlockSpec((1, F), lambda g: (0, 0)),
        ],
        out_specs=pl.BlockSpec(memory_space=pltpu.MemorySpace.HBM),
        out_shape=jax.ShapeDtypeStruct((B, F, J), jnp.float32),
        scratch_shapes=[pltpu.VMEM((NBUF, F, J), jnp.float32)]
        + [pltpu.SemaphoreType.DMA] * (NBUF * NSPL),
    )(xp, yp, vxp, vyp, fx, fy)


def kernel(frame):
    xp = frame[:, :, 1:2]
    yp = frame[:, :, 2:3]
    vxp = frame[:, :, 3:4]
    vyp = frame[:, :, 4:5]
    fx_np, fy_np = _field_xy()
    return _run(xp, yp, vxp, vyp, jnp.asarray(fx_np), jnp.asarray(fy_np))
